# baseline (speedup 1.0000x reference)
import functools

import jax
import jax.numpy as jnp
from jax import lax
from jax.experimental import pallas as pl
from jax.experimental.pallas import tpu as pltpu
from jax.experimental.pallas import tpu_sc as plsc

B = 16384
F = 26
VOCAB = 100000
D = 16

_STRIP = 0

NC = 2
NS = 16
NW = NC * NS
SAMP = B // NW
PIECE = 128
CH = 128
NCHUNK = SAMP // CH
PIECES_PER_CHUNK = CH * F // PIECE
PIECES_PER_WORKER = SAMP * F // PIECE
ROWS_PER_CHUNK = CH * F


def _floop(n, body):
    lax.fori_loop(0, n, lambda i, c: (body(i), c)[1], None)


def _sc_fm_tvec(idx3, table):
    mesh = plsc.VectorSubcoreMesh(core_axis_name="c", subcore_axis_name="s")

    @functools.partial(
        pl.kernel,
        mesh=mesh,
        compiler_params=pltpu.CompilerParams(use_tc_tiling_on_sc=False),
        out_type=jax.ShapeDtypeStruct((B, D), jnp.float32),
        scratch_types=[
            pltpu.VMEM((PIECES_PER_WORKER, PIECE), jnp.int32),
            pltpu.VMEM((ROWS_PER_CHUNK, D), jnp.float32),
            pltpu.VMEM((SAMP, D), jnp.float32),
            pltpu.SemaphoreType.DMA,
        ],
    )
    def k(idx_hbm, table_hbm, out_hbm, idx_v, rows_v, out_v, sem):
        wid = lax.axis_index("s") * NC + lax.axis_index("c")
        iota = lax.iota(jnp.int32, 16)

        pltpu.sync_copy(idx_hbm.at[pl.ds(wid * PIECES_PER_WORKER,
                                         PIECES_PER_WORKER)], idx_v)

        def off_row(r):
            for kk in range(PIECE // 16):
                col = kk * 16
                pos = iota + (r * PIECE + col)
                f = lax.rem(pos, F)
                idx_v[r, pl.ds(col, 16)] = (idx_v[r, pl.ds(col, 16)]
                                            + f * VOCAB)
        _floop(PIECES_PER_WORKER, off_row)

        def chunk(c):
            pbase = c * PIECES_PER_CHUNK

            def fire(p):
                pltpu.make_async_copy(
                    table_hbm.at[idx_v.at[pbase + p]],
                    rows_v.at[pl.ds(p * PIECE, PIECE)],
                    sem,
                ).start()
            _floop(PIECES_PER_CHUNK, fire)

            def drain(p):
                pltpu.make_async_copy(
                    table_hbm.at[idx_v.at[pbase + p]],
                    rows_v.at[pl.ds(p * PIECE, PIECE)],
                    sem,
                ).wait()
            _floop(PIECES_PER_CHUNK, drain)

            def sample(s):
                base = s * F
                r0 = rows_v[base, :]
                acc = r0
                sumsq = r0 * r0
                for f in range(1, F):
                    r = rows_v[base + f, :]
                    acc = acc + r
                    sumsq = sumsq + r * r
                out_v[c * CH + s, :] = 0.5 * (acc * acc - sumsq)
            _floop(CH, sample)
        if _STRIP == 0:
            _floop(NCHUNK, chunk)

        pltpu.sync_copy(out_v, out_hbm.at[pl.ds(wid * SAMP, SAMP)])

    return k(idx3, table)


def _head(t_ref, label_ref, y_ref, loss_ref):
    x = t_ref[...]
    sel = (jnp.arange(128, dtype=jnp.int32)[:, None] // D
           == jnp.arange(8, dtype=jnp.int32)[None, :]).astype(jnp.float32)
    logit = jnp.dot(x, sel, preferred_element_type=jnp.float32)
    y = 1.0 / (1.0 + jnp.exp(-logit))
    y_ref[...] = y
    eps = 1e-7
    p = jnp.clip(y, eps, 1.0 - eps)
    lab = label_ref[...]
    terms = lab * jnp.log(p) + (1.0 - lab) * jnp.log(1.0 - p)
    loss_ref[0, 0] = -jnp.sum(terms) / B


def _tc_head(tvec, label):
    y2, loss2 = pl.pallas_call(
        _head,
        out_shape=[
            jax.ShapeDtypeStruct((B // 8, 8), jnp.float32),
            jax.ShapeDtypeStruct((1, 1), jnp.float32),
        ],
        out_specs=[
            pl.BlockSpec(memory_space=pltpu.VMEM),
            pl.BlockSpec(memory_space=pltpu.SMEM),
        ],
    )(tvec.reshape(B // 8, 8 * D), label.reshape(B // 8, 8))
    return y2, loss2


_TCT = 8192
_NTBLK = -(-(F * VOCAB) // _TCT)


def _tc_transpose_body(in_ref, out_ref):
    x = in_ref[...]
    out_ref[...] = x.T


def _tc_transpose(tableT):
    return pl.pallas_call(
        _tc_transpose_body,
        grid=(_NTBLK,),
        in_specs=[pl.BlockSpec((D, _TCT), lambda i: (0, i))],
        out_specs=pl.BlockSpec((_TCT, D), lambda i: (i, 0)),
        out_shape=jax.ShapeDtypeStruct((F * VOCAB, D), jnp.float32),
    )(tableT)


def _sc_noop(idx3):
    mesh = plsc.VectorSubcoreMesh(core_axis_name="c", subcore_axis_name="s")

    @functools.partial(
        pl.kernel,
        mesh=mesh,
        compiler_params=pltpu.CompilerParams(use_tc_tiling_on_sc=False),
        out_type=jax.ShapeDtypeStruct((B, D), jnp.float32),
        scratch_types=[
            pltpu.VMEM((PIECES_PER_WORKER, PIECE), jnp.int32),
            pltpu.VMEM((SAMP, D), jnp.float32),
        ],
    )
    def k(idx_hbm, out_hbm, idx_v, out_v):
        wid = lax.axis_index("s") * NC + lax.axis_index("c")
        pltpu.sync_copy(idx_hbm.at[pl.ds(wid * PIECES_PER_WORKER,
                                         PIECES_PER_WORKER)], idx_v)
        pltpu.sync_copy(out_v, out_hbm.at[pl.ds(wid * SAMP, SAMP)])

    return k(idx3)


def kernel(indices, label, table):
    idx3 = indices.astype(jnp.int32).reshape(B * F // PIECE, PIECE)
    if _STRIP == 2:
        tvec = _sc_noop(idx3)
    else:
        table_rm = _tc_transpose(table.T)
        tvec = _sc_fm_tvec(idx3, table_rm)
    y2, loss2 = _tc_head(tvec, label)
    return y2.reshape(B, 1), loss2[0, 0]


# speedup vs baseline: 4.0170x; 4.0170x over previous
import functools

import jax
import jax.numpy as jnp
from jax import lax
from jax.experimental import pallas as pl
from jax.experimental.pallas import tpu as pltpu
from jax.experimental.pallas import tpu_sc as plsc

B = 16384
F = 26
VOCAB = 100000
D = 16

NC = 2
NS = 16
NW = NC * NS
SAMP = B // NW
PIECE = 128
CH = 128
NCHUNK = SAMP // CH
PIECES_PER_CHUNK = CH * F // PIECE
PIECES_PER_WORKER = SAMP * F // PIECE
ROWS_PER_CHUNK = CH * F


def _floop(n, body):
    lax.fori_loop(0, n, lambda i, c: (body(i), c)[1], None)


def _sc_fm_tvec(idx3, table):
    mesh = plsc.VectorSubcoreMesh(core_axis_name="c", subcore_axis_name="s")

    @functools.partial(
        pl.kernel,
        mesh=mesh,
        compiler_params=pltpu.CompilerParams(use_tc_tiling_on_sc=False),
        out_type=jax.ShapeDtypeStruct((B, D), jnp.float32),
        scratch_types=[
            pltpu.VMEM((PIECES_PER_WORKER, PIECE), jnp.int32),
            pltpu.VMEM((ROWS_PER_CHUNK, D), jnp.float32),
            pltpu.VMEM((SAMP, D), jnp.float32),
            pltpu.SemaphoreType.DMA,
        ],
    )
    def k(idx_hbm, table_hbm, out_hbm, idx_v, rows_v, out_v, sem):
        wid = lax.axis_index("s") * NC + lax.axis_index("c")
        iota = lax.iota(jnp.int32, 16)

        pltpu.sync_copy(idx_hbm.at[pl.ds(wid * PIECES_PER_WORKER,
                                         PIECES_PER_WORKER)], idx_v)

        def off_row(r):
            for kk in range(PIECE // 16):
                col = kk * 16
                pos = iota + (r * PIECE + col)
                f = lax.rem(pos, F)
                idx_v[r, pl.ds(col, 16)] = (idx_v[r, pl.ds(col, 16)]
                                            + f * VOCAB)
        _floop(PIECES_PER_WORKER, off_row)

        def chunk(c):
            pbase = c * PIECES_PER_CHUNK

            def fire(p):
                pltpu.make_async_copy(
                    table_hbm.at[idx_v.at[pbase + p]],
                    rows_v.at[pl.ds(p * PIECE, PIECE)],
                    sem,
                ).start()
            _floop(PIECES_PER_CHUNK, fire)

            def drain(p):
                pltpu.make_async_copy(
                    table_hbm.at[idx_v.at[pbase + p]],
                    rows_v.at[pl.ds(p * PIECE, PIECE)],
                    sem,
                ).wait()
            _floop(PIECES_PER_CHUNK, drain)

            def sample(s):
                base = s * F
                r0 = rows_v[base, :]
                acc = r0
                sumsq = r0 * r0
                for f in range(1, F):
                    r = rows_v[base + f, :]
                    acc = acc + r
                    sumsq = sumsq + r * r
                out_v[c * CH + s, :] = 0.5 * (acc * acc - sumsq)
            _floop(CH, sample)
        _floop(NCHUNK, chunk)

        pltpu.sync_copy(out_v, out_hbm.at[pl.ds(wid * SAMP, SAMP)])

    return k(idx3, table)


def _head(t_ref, label_ref, y_ref, loss_ref):
    x = t_ref[...]
    sel = (jnp.arange(128, dtype=jnp.int32)[:, None] // D
           == jnp.arange(8, dtype=jnp.int32)[None, :]).astype(jnp.float32)
    logit = jnp.dot(x, sel, preferred_element_type=jnp.float32)
    y = 1.0 / (1.0 + jnp.exp(-logit))
    y_ref[...] = y
    eps = 1e-7
    p = jnp.clip(y, eps, 1.0 - eps)
    lab = label_ref[...]
    terms = lab * jnp.log(p) + (1.0 - lab) * jnp.log(1.0 - p)
    loss_ref[0, 0] = -jnp.sum(terms) / B


def _tc_head(tvec, label):
    y2, loss2 = pl.pallas_call(
        _head,
        out_shape=[
            jax.ShapeDtypeStruct((B // 8, 8), jnp.float32),
            jax.ShapeDtypeStruct((1, 1), jnp.float32),
        ],
        out_specs=[
            pl.BlockSpec(memory_space=pltpu.VMEM),
            pl.BlockSpec(memory_space=pltpu.SMEM),
        ],
    )(tvec.reshape(B // 8, 8 * D), label.reshape(B // 8, 8))
    return y2, loss2


_N = F * VOCAB
_CC = 1024
_NFULL = _N // _CC
_NMAX = -(-_NFULL // NW)


def _sc_transpose(tableT, tail_pad):
    mesh = plsc.VectorSubcoreMesh(core_axis_name="c", subcore_axis_name="s")

    @functools.partial(
        pl.kernel,
        mesh=mesh,
        compiler_params=pltpu.CompilerParams(use_tc_tiling_on_sc=True,
                                             needs_layout_passes=False),
        out_type=jax.ShapeDtypeStruct((_N * D,), jnp.float32),
        scratch_types=[
            pltpu.VMEM((D, _CC + 81), jnp.float32),
            pltpu.VMEM((D, _CC + 81), jnp.float32),
            pltpu.VMEM((_CC * D,), jnp.float32),
            pltpu.VMEM((_CC * D,), jnp.float32),
            pltpu.SemaphoreType.DMA,
            pltpu.SemaphoreType.DMA,
            pltpu.SemaphoreType.DMA,
            pltpu.SemaphoreType.DMA,
        ],
    )
    def k(tt_hbm, tail_hbm, out_hbm, in0, in1, ob0, ob1,
          rs0, rs1, ws0, ws1):
        wid = lax.axis_index("s") * NC + lax.axis_index("c")
        iota = lax.iota(jnp.int32, 16)
        sidx = [iota * D + d for d in range(D)]
        ins, obs = (in0, in1), (ob0, ob1)
        rsems, wsems = (rs0, rs1), (ws0, ws1)

        def cid(i):
            return i * NW + wid

        def rd(i, b):
            return pltpu.make_async_copy(
                tt_hbm.at[:, pl.ds(cid(i) * _CC, _CC)],
                ins[b].at[:, pl.ds(0, _CC)], rsems[b])

        def wr(i, b):
            return pltpu.make_async_copy(
                obs[b], out_hbm.at[pl.ds(cid(i) * _CC * D, _CC * D)],
                wsems[b])

        def transpose_grps(in_v, out_v, ngrp):
            def colgrp(g):
                sub = out_v.at[pl.ds(g * 16 * D, 16 * D)]
                for d in range(D):
                    v = in_v[d, pl.ds(g * 16, 16)]
                    plsc.store_scatter(sub, [sidx[d]], v)
            lax.fori_loop(0, ngrp, lambda g, c: (colgrp(g), c)[1], None)

        for b in range(2):
            @pl.when(cid(b) < _NFULL)
            def _():
                rd(b, b).start()

        def body(j):
            for b in range(2):
                i = 2 * j + b

                @pl.when((i >= 2) & (cid(i - 2) < _NFULL))
                def _():
                    wr(i - 2, b).wait()

                @pl.when(cid(i) < _NFULL)
                def _():
                    rd(i, b).wait()
                    transpose_grps(ins[b], obs[b], _CC // 16)
                    wr(i, b).start()

                @pl.when(cid(i + 2) < _NFULL)
                def _():
                    rd(i + 2, b).start()
        lax.fori_loop(0, _NMAX // 2, lambda j, c: (body(j), c)[1], None)

        for b in range(2):
            i = _NMAX - 2 + b

            @pl.when(cid(i) < _NFULL)
            def _():
                wr(i, b).wait()

        @pl.when(wid == NW - 1)
        def _tail():
            pltpu.sync_copy(tail_hbm, in0.at[:, pl.ds(0, 128)])
            transpose_grps(in0, ob0, 4)
            pltpu.sync_copy(ob0.at[pl.ds(0, 64 * D)],
                            out_hbm.at[pl.ds((_N - 64) * D, 64 * D)])

    return k(tableT, tail_pad)


def kernel(indices, label, table):
    idx3 = indices.astype(jnp.int32).reshape(B * F // PIECE, PIECE)
    tableT = table.T
    tail_pad = jnp.pad(tableT[:, _N - 64:], ((0, 0), (0, 64)))
    table_rm = _sc_transpose(tableT, tail_pad).reshape(F * VOCAB, D)
    tvec = _sc_fm_tvec(idx3, table_rm)
    y2, loss2 = _tc_head(tvec, label)
    return y2.reshape(B, 1), loss2[0, 0]
